# baseline (speedup 1.0000x reference)
import functools

import jax
import jax.numpy as jnp
from jax.experimental import pallas as pl
from jax.experimental.pallas import tpu as pltpu

N_BLK = 400


def _h0_body(x_ref, w0_ref, b0_ref, h0_ref):
    acc = jnp.dot(x_ref[...].astype(jnp.bfloat16), w0_ref[...].astype(jnp.bfloat16),
                  preferred_element_type=jnp.float32)
    h0_ref[...] = (acc + b0_ref[...]).astype(jnp.bfloat16)


def _layer1_body(adj_ref, h0_ref, w1_ref, b1_ref, z_ref):
    a = adj_ref[...].astype(jnp.bfloat16)
    acc = jnp.dot(a, h0_ref[...], preferred_element_type=jnp.float32)
    h1 = jnp.maximum(acc, 0.0).astype(jnp.bfloat16)
    z = jnp.dot(h1, w1_ref[...], preferred_element_type=jnp.float32) + b1_ref[...]
    z_ref[...] = z.astype(jnp.bfloat16)


def _layer2_body(adj_ref, z_ref, out_ref):
    a = adj_ref[...].astype(jnp.bfloat16)
    out_ref[...] = jnp.dot(a, z_ref[...], preferred_element_type=jnp.float32)


@functools.partial(jax.jit, static_argnums=())
def kernel(x, adj, W0, b0, W1, b1):
    n, f = x.shape
    h = W0.shape[1]
    c = W1.shape[1]
    nblk = n // N_BLK

    h0 = pl.pallas_call(
        _h0_body,
        out_shape=jax.ShapeDtypeStruct((n, h), jnp.bfloat16),
    )(x, W0, b0.reshape(1, h))

    z = pl.pallas_call(
        _layer1_body,
        grid=(nblk,),
        in_specs=[
            pl.BlockSpec((N_BLK, n), lambda i: (i, 0)),
            pl.BlockSpec((n, h), lambda i: (0, 0)),
            pl.BlockSpec((h, c), lambda i: (0, 0)),
            pl.BlockSpec((1, c), lambda i: (0, 0)),
        ],
        out_specs=pl.BlockSpec((N_BLK, c), lambda i: (i, 0)),
        out_shape=jax.ShapeDtypeStruct((n, c), jnp.bfloat16),
    )(adj, h0, W1.astype(jnp.bfloat16), b1.reshape(1, c))

    out = pl.pallas_call(
        _layer2_body,
        grid=(nblk,),
        in_specs=[
            pl.BlockSpec((N_BLK, n), lambda i: (i, 0)),
            pl.BlockSpec((n, c), lambda i: (0, 0)),
        ],
        out_specs=pl.BlockSpec((N_BLK, c), lambda i: (i, 0)),
        out_shape=jax.ShapeDtypeStruct((n, c), jnp.float32),
    )(adj, z)

    return out


# speedup vs baseline: 1.0217x; 1.0217x over previous
import functools

import jax
import jax.numpy as jnp
from jax.experimental import pallas as pl
from jax.experimental.pallas import tpu as pltpu

BM = 200
K_CACHE = 8


def _h0_body(x_ref, w0_ref, b0_ref, h0_ref):
    acc = jnp.dot(x_ref[...].astype(jnp.bfloat16), w0_ref[...].astype(jnp.bfloat16),
                  preferred_element_type=jnp.float32)
    h0_ref[...] = (acc + b0_ref[...]).astype(jnp.bfloat16)


def _fused_body(adj_ref, h0_ref, w1_ref, b1_ref, out_ref, z_ref, cache_ref,
                *, nblk, kcache):
    p = pl.program_id(0)
    i = pl.program_id(1)

    @pl.when(p == 0)
    def _phase0():
        a = adj_ref[...].astype(jnp.bfloat16)
        acc = jnp.dot(a, h0_ref[...], preferred_element_type=jnp.float32)
        h1 = jnp.maximum(acc, 0.0).astype(jnp.bfloat16)
        z = jnp.dot(h1, w1_ref[...], preferred_element_type=jnp.float32) + b1_ref[...]
        z_ref[pl.ds(i * BM, BM), :] = z.astype(jnp.bfloat16)

        @pl.when(i < kcache)
        def _():
            cache_ref[pl.ds(i * BM, BM), :] = a

    @pl.when(p == 1)
    def _phase1():
        z = z_ref[...]
        use_cache = jnp.logical_and(i >= 1, i <= kcache)

        @pl.when(use_cache)
        def _():
            idx = jnp.maximum(i - 1, 0)
            a = cache_ref[pl.ds(idx * BM, BM), :]
            out_ref[...] = jnp.dot(a, z, preferred_element_type=jnp.float32)

        @pl.when(jnp.logical_not(use_cache))
        def _():
            a = adj_ref[...].astype(jnp.bfloat16)
            out_ref[...] = jnp.dot(a, z, preferred_element_type=jnp.float32)


@jax.jit
def kernel(x, adj, W0, b0, W1, b1):
    n, f = x.shape
    h = W0.shape[1]
    c = W1.shape[1]
    nblk = n // BM
    kcache = min(K_CACHE, nblk - 2)

    h0 = pl.pallas_call(
        _h0_body,
        out_shape=jax.ShapeDtypeStruct((n, h), jnp.bfloat16),
    )(x, W0, b0.reshape(1, h))

    def adj_map(p, i):
        return (jnp.where(p == 0, i,
                          jnp.where(i <= kcache, nblk - 1, i - 1)), 0)

    def out_map(p, i):
        return (jnp.where(p == 0, nblk,
                          jnp.where(i == 0, nblk - 1, i - 1)), 0)

    out = pl.pallas_call(
        functools.partial(_fused_body, nblk=nblk, kcache=kcache),
        grid=(2, nblk),
        in_specs=[
            pl.BlockSpec((BM, n), adj_map),
            pl.BlockSpec((n, h), lambda p, i: (0, 0)),
            pl.BlockSpec((h, c), lambda p, i: (0, 0)),
            pl.BlockSpec((1, c), lambda p, i: (0, 0)),
        ],
        out_specs=pl.BlockSpec((BM, c), out_map),
        out_shape=jax.ShapeDtypeStruct((n + BM, c), jnp.float32),
        scratch_shapes=[
            pltpu.VMEM((n, c), jnp.bfloat16),
            pltpu.VMEM((kcache * BM, n), jnp.bfloat16),
        ],
        compiler_params=pltpu.CompilerParams(vmem_limit_bytes=64 * 1024 * 1024),
    )(adj, h0, W1.astype(jnp.bfloat16), b1.reshape(1, c))

    return out[:n]


# speedup vs baseline: 1.0343x; 1.0124x over previous
import functools

import jax
import jax.numpy as jnp
from jax.experimental import pallas as pl
from jax.experimental.pallas import tpu as pltpu

BM = 200
SUB_ROWS = (48, 48, 48, 56)
SUB_OFF = (0, 48, 96, 144)
NBUF = 4
DEPTH = 3
K_CACHE = 4


def _h0_body(x_ref, w0_ref, b0_ref, h0_ref):
    acc = jnp.dot(x_ref[...].astype(jnp.bfloat16), w0_ref[...].astype(jnp.bfloat16),
                  preferred_element_type=jnp.float32)
    h0_ref[...] = (acc + b0_ref[...]).astype(jnp.bfloat16)


def _fused_body(adj_hbm, h0_ref, w1_ref, b1_ref, out_ref,
                bufs_ref, z_ref, zbf_ref, cache_ref, sems,
                *, nblk, kcache):
    t = pl.program_id(0)

    def fetch_block(step):
        s1 = step - nblk
        return jnp.where(step < nblk, step,
                         jnp.where(s1 < kcache, -1, s1))

    def issue(step):
        b = fetch_block(step)

        @pl.when(jnp.logical_and(b >= 0, step < 2 * nblk))
        def _():
            slot = jax.lax.rem(step, NBUF)
            bb = jnp.maximum(b, 0)
            for off, rows in zip(SUB_OFF, SUB_ROWS):
                pltpu.make_async_copy(
                    adj_hbm.at[pl.ds(bb * BM + off, rows), :],
                    bufs_ref.at[slot, pl.ds(off, rows), :],
                    sems.at[slot],
                ).start()

    @pl.when(t == 0)
    def _prologue():
        for d in range(DEPTH):
            issue(d)

    issue(t + DEPTH)

    def wait_block(step):
        slot = jax.lax.rem(step, NBUF)
        b = jnp.maximum(fetch_block(step), 0)
        for off, rows in zip(SUB_OFF, SUB_ROWS):
            pltpu.make_async_copy(
                adj_hbm.at[pl.ds(b * BM + off, rows), :],
                bufs_ref.at[slot, pl.ds(off, rows), :],
                sems.at[slot],
            ).wait()
        return slot

    @pl.when(t < nblk)
    def _phase0():
        slot = wait_block(t)
        a = bufs_ref[slot].astype(jnp.bfloat16)
        acc = jnp.dot(a, h0_ref[...], preferred_element_type=jnp.float32)
        h1 = jnp.maximum(acc, 0.0).astype(jnp.bfloat16)
        z = jnp.dot(h1, w1_ref[...], preferred_element_type=jnp.float32) + b1_ref[...]
        z_ref[pl.ds(t * BM, BM), :] = z

        @pl.when(t < kcache)
        def _():
            cache_ref[jnp.minimum(t, kcache - 1)] = a

    @pl.when(t == nblk)
    def _cast_z():
        zbf_ref[...] = z_ref[...].astype(jnp.bfloat16)

    @pl.when(t >= nblk)
    def _phase1():
        s = t - nblk
        z = zbf_ref[...]

        @pl.when(s < kcache)
        def _():
            a = cache_ref[jnp.minimum(s, kcache - 1)]
            out_ref[...] = jnp.dot(a, z, preferred_element_type=jnp.float32)

        @pl.when(s >= kcache)
        def _():
            slot = wait_block(t)
            a = bufs_ref[slot].astype(jnp.bfloat16)
            out_ref[...] = jnp.dot(a, z, preferred_element_type=jnp.float32)


@jax.jit
def kernel(x, adj, W0, b0, W1, b1):
    n, f = x.shape
    h = W0.shape[1]
    c = W1.shape[1]
    nblk = n // BM
    kcache = min(K_CACHE, nblk - 1)

    h0 = pl.pallas_call(
        _h0_body,
        out_shape=jax.ShapeDtypeStruct((n, h), jnp.bfloat16),
    )(x, W0, b0.reshape(1, h))

    def out_map(t):
        return (jnp.where(t < nblk, nblk, t - nblk), 0)

    out = pl.pallas_call(
        functools.partial(_fused_body, nblk=nblk, kcache=kcache),
        grid=(2 * nblk,),
        in_specs=[
            pl.BlockSpec(memory_space=pltpu.MemorySpace.HBM),
            pl.BlockSpec((n, h), lambda t: (0, 0)),
            pl.BlockSpec((h, c), lambda t: (0, 0)),
            pl.BlockSpec((1, c), lambda t: (0, 0)),
        ],
        out_specs=pl.BlockSpec((BM, c), out_map),
        out_shape=jax.ShapeDtypeStruct((n + BM, c), jnp.float32),
        scratch_shapes=[
            pltpu.VMEM((NBUF, BM, n), jnp.float32),
            pltpu.VMEM((n, c), jnp.float32),
            pltpu.VMEM((n, c), jnp.bfloat16),
            pltpu.VMEM((kcache, BM, n), jnp.bfloat16),
            pltpu.SemaphoreType.DMA((NBUF,)),
        ],
        compiler_params=pltpu.CompilerParams(vmem_limit_bytes=64 * 1024 * 1024),
    )(adj, h0, W1.astype(jnp.bfloat16), b1.reshape(1, c))

    return out[:n]
